# Initial kernel scaffold; baseline (speedup 1.0000x reference)
import jax
import jax.numpy as jnp
from jax.experimental import pallas as pl


def kernel(input_ids, W, pe):
    raise NotImplementedError("write your pallas kernel here")


# speedup vs baseline: 1.2927x; 1.2927x over previous
import functools

import jax
import jax.numpy as jnp
from jax import lax
from jax.experimental import pallas as pl
from jax.experimental.pallas import tpu as pltpu
from jax.experimental.pallas import tpu_sc as plsc

B = 4096
L = 200
DIM = 64
WPAD = 128
NPE = 208
NW = 32
RPW = B // NW
RW = L * DIM


def _sc_embed(ids_hbm, w_hbm, pe_hbm, emb_hbm, word_hbm, pos_hbm,
              pe_v, idx_v, wrow_v, word_v, pos_v, emb_v, sem):
    wid = lax.axis_index("s") * 2 + lax.axis_index("c")
    base = wid * RPW

    pltpu.sync_copy(pe_hbm.at[pl.ds(0, NPE * DIM)], pe_v)
    pltpu.sync_copy(ids_hbm.at[pl.ds(base * L, RPW * L)], idx_v)

    lane = lax.iota(jnp.int32, 16)
    tail_mask = lane >= 8

    def row(i, carry):
        r = base + i
        off = i * L

        nz = jnp.zeros((16,), jnp.int32)
        for k in range(L // 16):
            v = idx_v[pl.ds(off + k * 16, 16)]
            nz = nz + plsc.all_reduce_population_count(v != 0)
        v = idx_v[pl.ds(off + L - 16, 16)]
        nz = nz + plsc.all_reduce_population_count((v != 0) & tail_mask)

        c0 = pltpu.async_copy(w_hbm.at[idx_v.at[pl.ds(off, 128)]],
                              wrow_v.at[pl.ds(0, 128)], sem)
        c1 = pltpu.async_copy(w_hbm.at[idx_v.at[pl.ds(off + 128, L - 128)]],
                              wrow_v.at[pl.ds(128, L - 128)], sem)
        c0.wait()
        c1.wait()

        def pos_row(j, _):
            ridx = jnp.where(j + 1 <= nz, (j + 1) * DIM, 0).astype(jnp.int32)
            for c in range(DIM // 16):
                col = lane + c * 16
                pvec = plsc.load_gather(pe_v, [ridx + col])
                wvec = wrow_v[j, pl.ds(c * 16, 16)]
                word_v[pl.ds(j * DIM + c * 16, 16)] = wvec
                pos_v[pl.ds(j * DIM + c * 16, 16)] = pvec
                emb_v[pl.ds(j * DIM + c * 16, 16)] = wvec + pvec
            return 0

        lax.fori_loop(0, L, pos_row, 0)

        pltpu.sync_copy(word_v, word_hbm.at[pl.ds(r * RW, RW)])
        pltpu.sync_copy(pos_v, pos_hbm.at[pl.ds(r * RW, RW)])
        pltpu.sync_copy(emb_v, emb_hbm.at[pl.ds(r * RW, RW)])
        return carry

    lax.fori_loop(0, RPW, row, 0)


def kernel(input_ids, W, pe):
    mesh = plsc.VectorSubcoreMesh(core_axis_name="c", subcore_axis_name="s")
    out = jax.ShapeDtypeStruct((B * L * DIM,), jnp.float32)
    f = functools.partial(
        pl.kernel,
        mesh=mesh,
        out_type=(out, out, out),
        compiler_params=pltpu.CompilerParams(needs_layout_passes=False),
        scratch_types=[
            pltpu.VMEM((NPE * DIM,), jnp.float32),
            pltpu.VMEM((RPW * L,), jnp.int32),
            pltpu.VMEM((L, WPAD), jnp.float32),
            pltpu.VMEM((RW,), jnp.float32),
            pltpu.VMEM((RW,), jnp.float32),
            pltpu.VMEM((RW,), jnp.float32),
            pltpu.SemaphoreType.DMA,
        ],
    )(_sc_embed)
    w_pad = jnp.pad(W, ((0, 0), (0, WPAD - DIM)))
    emb, word, pos = f(input_ids.reshape(B * L), w_pad, pe.reshape(-1))
    shape = (B, L, DIM)
    return emb.reshape(shape), word.reshape(shape), pos.reshape(shape)
